# Initial kernel scaffold; baseline (speedup 1.0000x reference)
import functools

import jax
import jax.numpy as jnp
from jax import lax
from jax.experimental import pallas as pl
from jax.experimental.pallas import tpu as pltpu
from jax.experimental.pallas import tpu_sc as plsc

N = 5000
C = 80
THR = 0.2
IOU_THR = 0.2
K = 100

L = 16
NS = 16
ROWS = 320
NV = ROWS // L
NP = NS * ROWS
NEG = jnp.float32(-1e30)
BIG = jnp.int32(1 << 30)
BO_PAD = 448
SC_PAD = 112


def _nms_kernel(sc_hbm, x1_hbm, y1_hbm, x2_hbm, y2_hbm,
                bo_hbm, so_hbm, lo_hbm,
                scv, x1v, y1v, x2v, y2v, arv, candv, labv,
                pubv, rbv, obv, osv, olv, shared):
    sid = lax.axis_index("s")
    base = sid * ROWS
    lanes = lax.iota(jnp.int32, L)
    zf = jnp.zeros((L,), jnp.float32)
    zi = jnp.zeros((L,), jnp.int32)

    pltpu.sync_copy(sc_hbm.at[pl.ds(base * C, ROWS * C)], scv)
    pltpu.sync_copy(x1_hbm.at[pl.ds(base, ROWS)], x1v)
    pltpu.sync_copy(y1_hbm.at[pl.ds(base, ROWS)], y1v)
    pltpu.sync_copy(x2_hbm.at[pl.ds(base, ROWS)], x2v)
    pltpu.sync_copy(y2_hbm.at[pl.ds(base, ROWS)], y2v)

    def zero_b(j, _):
        obv[pl.ds(j * L, L)] = zf
        return 0
    lax.fori_loop(0, BO_PAD // L, zero_b, 0)

    def zero_s(j, _):
        osv[pl.ds(j * L, L)] = zf
        olv[pl.ds(j * L, L)] = zi
        return 0
    lax.fori_loop(0, SC_PAD // L, zero_s, 0)

    def grp(g, _):
        row_off = g * (L * C) + lanes * C

        def cls(c, MA):
            M, A = MA
            v = plsc.load_gather(scv, [row_off + c])
            take = v > M
            return (jnp.where(take, v, M),
                    jnp.where(take, jnp.full((L,), c, jnp.int32), A))

        M, A = lax.fori_loop(0, C, cls, (jnp.full((L,), NEG, jnp.float32), zi))
        sl = pl.ds(g * L, L)
        candv[sl] = jnp.where(M > THR, M, jnp.full((L,), NEG, jnp.float32))
        labv[sl] = A
        arv[sl] = (x2v[sl] - x1v[sl]) * (y2v[sl] - y1v[sl])
        return 0
    lax.fori_loop(0, NV, grp, 0)

    def cond(st):
        _, cont = st
        return cont

    def body(st):
        k, _ = st

        def scan_j(j, bSI):
            bS, bI = bSI
            sl = pl.ds(j * L, L)
            v = candv[sl]
            gi = base + j * L + lanes
            take = v > bS
            return (jnp.where(take, v, bS), jnp.where(take, gi, bI))

        bS, bI = lax.fori_loop(
            0, NV, scan_j,
            (jnp.full((L,), NEG, jnp.float32), jnp.full((L,), BIG, jnp.int32)))
        Ml = jnp.max(bS)
        il = jnp.min(jnp.where(bS == Ml, bI, jnp.full((L,), BIG, jnp.int32)))
        li = jnp.clip(il - base, 0, ROWS - 1)
        liv = jnp.full((L,), li, jnp.int32)
        cx1 = plsc.load_gather(x1v, [liv])
        cy1 = plsc.load_gather(y1v, [liv])
        cx2 = plsc.load_gather(x2v, [liv])
        cy2 = plsc.load_gather(y2v, [liv])
        car = plsc.load_gather(arv, [liv])
        clb = plsc.load_gather(labv, [liv])

        pub = jnp.full((L,), Ml, jnp.float32)
        pub = jnp.where(lanes == 1,
                        plsc.bitcast(jnp.full((L,), il, jnp.int32),
                                     jnp.float32), pub)
        pub = jnp.where(lanes == 2, cx1, pub)
        pub = jnp.where(lanes == 3, cy1, pub)
        pub = jnp.where(lanes == 4, cx2, pub)
        pub = jnp.where(lanes == 5, cy2, pub)
        pub = jnp.where(lanes == 6, plsc.bitcast(clb, jnp.float32), pub)
        pub = jnp.where(lanes == 7, car, pub)
        pubv[...] = pub
        pltpu.sync_copy(pubv, shared.at[sid])
        plsc.subcore_barrier()
        pltpu.sync_copy(shared, rbv)
        plsc.subcore_barrier()

        def col(c):
            return plsc.load_gather(rbv, [lanes, jnp.full((L,), c, jnp.int32)])

        s_all = col(0)
        i_all = plsc.bitcast(col(1), jnp.int32)
        M = jnp.max(s_all)
        cont2 = M > jnp.float32(-1e29)
        iw = jnp.min(jnp.where(s_all == M, i_all,
                               jnp.full((L,), BIG, jnp.int32)))
        wm = i_all == iw
        negv = jnp.full((L,), NEG, jnp.float32)
        X1 = jnp.max(jnp.where(wm, col(2), negv))
        Y1 = jnp.max(jnp.where(wm, col(3), negv))
        X2 = jnp.max(jnp.where(wm, col(4), negv))
        Y2 = jnp.max(jnp.where(wm, col(5), negv))
        LB = jnp.max(jnp.where(wm, plsc.bitcast(col(6), jnp.int32),
                               jnp.full((L,), -1, jnp.int32)))
        WA = jnp.max(jnp.where(wm, col(7), negv))

        X1v = jnp.full((L,), X1, jnp.float32)
        Y1v = jnp.full((L,), Y1, jnp.float32)
        X2v = jnp.full((L,), X2, jnp.float32)
        Y2v = jnp.full((L,), Y2, jnp.float32)
        WAv = jnp.full((L,), WA, jnp.float32)
        iwv = jnp.full((L,), iw, jnp.int32)

        @pl.when(cont2)
        def _():
            def sup_j(j, _):
                sl = pl.ds(j * L, L)
                xx1 = jnp.maximum(x1v[sl], X1v)
                yy1 = jnp.maximum(y1v[sl], Y1v)
                xx2 = jnp.minimum(x2v[sl], X2v)
                yy2 = jnp.minimum(y2v[sl], Y2v)
                inter = (jnp.maximum(xx2 - xx1, 0.0) *
                         jnp.maximum(yy2 - yy1, 0.0))
                union = WAv + arv[sl] - inter
                iou = inter / (union + 1e-8)
                gi = base + j * L + lanes
                kill = (iou >= IOU_THR) | (gi == iwv)
                candv[sl] = jnp.where(kill, negv, candv[sl])
                return 0
            lax.fori_loop(0, NV, sup_j, 0)

        @pl.when(cont2 & (sid == 0))
        def _():
            bvals = X1v
            bvals = jnp.where(lanes == 1, Y1v, bvals)
            bvals = jnp.where(lanes == 2, X2v, bvals)
            bvals = jnp.where(lanes == 3, Y2v, bvals)
            plsc.store_scatter(obv, [4 * k + lanes], bvals, mask=lanes < 4)
            kv = jnp.full((L,), k, jnp.int32)
            plsc.store_scatter(osv, [kv], jnp.full((L,), M, jnp.float32),
                               mask=lanes == 0)
            plsc.store_scatter(olv, [kv], jnp.full((L,), LB, jnp.int32),
                               mask=lanes == 0)

        k2 = k + cont2.astype(jnp.int32)
        return (k2, cont2 & (k2 < K))

    lax.while_loop(cond, body, (jnp.int32(0), jnp.bool_(True)))

    @pl.when(sid == 0)
    def _():
        pltpu.sync_copy(obv, bo_hbm)
        pltpu.sync_copy(osv, so_hbm)
        pltpu.sync_copy(olv, lo_hbm)


@functools.partial(
    pl.kernel,
    out_type=(
        jax.ShapeDtypeStruct((BO_PAD,), jnp.float32),
        jax.ShapeDtypeStruct((SC_PAD,), jnp.float32),
        jax.ShapeDtypeStruct((SC_PAD,), jnp.int32),
    ),
    mesh=plsc.VectorSubcoreMesh(
        core_axis_name="c", subcore_axis_name="s", num_cores=1),
    scratch_types=[
        pltpu.VMEM((ROWS * C,), jnp.float32),
        pltpu.VMEM((ROWS,), jnp.float32),
        pltpu.VMEM((ROWS,), jnp.float32),
        pltpu.VMEM((ROWS,), jnp.float32),
        pltpu.VMEM((ROWS,), jnp.float32),
        pltpu.VMEM((ROWS,), jnp.float32),
        pltpu.VMEM((ROWS,), jnp.float32),
        pltpu.VMEM((ROWS,), jnp.int32),
        pltpu.VMEM((L,), jnp.float32),
        pltpu.VMEM((NS, L), jnp.float32),
        pltpu.VMEM((BO_PAD,), jnp.float32),
        pltpu.VMEM((SC_PAD,), jnp.float32),
        pltpu.VMEM((SC_PAD,), jnp.int32),
        pltpu.VMEM_SHARED((NS, L), jnp.float32),
    ],
)
def _nms_call(sc_hbm, x1_hbm, y1_hbm, x2_hbm, y2_hbm,
              bo_hbm, so_hbm, lo_hbm, *scratch):
    _nms_kernel(sc_hbm, x1_hbm, y1_hbm, x2_hbm, y2_hbm,
                bo_hbm, so_hbm, lo_hbm, *scratch)


@jax.jit
def kernel(boxes, scores):
    bp = jnp.pad(boxes, ((0, NP - N), (0, 0)))
    sp = jnp.pad(scores, ((0, NP - N), (0, 0)))
    bo, so, lo = _nms_call(
        sp.reshape(-1),
        bp[:, 0], bp[:, 1], bp[:, 2], bp[:, 3])
    return (bo[:4 * K].reshape(1, K, 4), so[:K][None], lo[:K][None])


# speedup vs baseline: 125.1194x; 125.1194x over previous
import functools

import jax
import jax.numpy as jnp
from jax import lax
from jax.experimental import pallas as pl
from jax.experimental.pallas import tpu as pltpu
from jax.experimental.pallas import tpu_sc as plsc

N = 5000
C = 80
THR = 0.2
IOU_THR = 0.2
K = 100

L = 16
NS = 16
ROWS = 320
NV = ROWS // L
NP = NS * ROWS
NEG = -1e30
BIG = 1 << 30
BO_PAD = 448
SC_PAD = 112


def _nms_kernel(sc_hbm, x1_hbm, y1_hbm, x2_hbm, y2_hbm,
                bo_hbm, so_hbm, lo_hbm,
                scv, x1v, y1v, x2v, y2v, arv, candv, labv,
                pubv, rbv, obv, osv, olv, shared):
    sid = lax.axis_index("s")
    base = sid * ROWS
    lanes = lax.iota(jnp.int32, L)
    zf = jnp.zeros((L,), jnp.float32)
    zi = jnp.zeros((L,), jnp.int32)

    pltpu.sync_copy(sc_hbm.at[pl.ds(base * C, ROWS * C)], scv)
    pltpu.sync_copy(x1_hbm.at[pl.ds(base, ROWS)], x1v)
    pltpu.sync_copy(y1_hbm.at[pl.ds(base, ROWS)], y1v)
    pltpu.sync_copy(x2_hbm.at[pl.ds(base, ROWS)], x2v)
    pltpu.sync_copy(y2_hbm.at[pl.ds(base, ROWS)], y2v)

    def zero_b(j, _):
        obv[pl.ds(j * L, L)] = zf
        return 0
    lax.fori_loop(0, BO_PAD // L, zero_b, 0)

    def zero_s(j, _):
        osv[pl.ds(j * L, L)] = zf
        olv[pl.ds(j * L, L)] = zi
        return 0
    lax.fori_loop(0, SC_PAD // L, zero_s, 0)

    def grp(g, _):
        row_off = g * (L * C) + lanes * C

        def cls(c, MA):
            M, A = MA
            v = plsc.load_gather(scv, [row_off + c])
            take = v > M
            return (jnp.where(take, v, M),
                    jnp.where(take, jnp.full((L,), c, jnp.int32), A))

        M, A = lax.fori_loop(0, C, cls, (jnp.full((L,), NEG, jnp.float32), zi))
        sl = pl.ds(g * L, L)
        candv[sl] = jnp.where(M > THR, M, jnp.full((L,), NEG, jnp.float32))
        labv[sl] = A
        arv[sl] = (x2v[sl] - x1v[sl]) * (y2v[sl] - y1v[sl])
        return 0
    lax.fori_loop(0, NV, grp, 0)

    def cond(st):
        _, cont = st
        return cont

    def body(st):
        k, _ = st

        def scan_j(j, bSI):
            bS, bI = bSI
            sl = pl.ds(j * L, L)
            v = candv[sl]
            gi = base + j * L + lanes
            take = v > bS
            return (jnp.where(take, v, bS), jnp.where(take, gi, bI))

        bS, bI = lax.fori_loop(
            0, NV, scan_j,
            (jnp.full((L,), NEG, jnp.float32), jnp.full((L,), BIG, jnp.int32)))
        Ml = jnp.max(bS)
        il = jnp.min(jnp.where(bS == Ml, bI, jnp.full((L,), BIG, jnp.int32)))
        li = jnp.clip(il - base, 0, ROWS - 1)
        liv = jnp.full((L,), li, jnp.int32)
        cx1 = plsc.load_gather(x1v, [liv])
        cy1 = plsc.load_gather(y1v, [liv])
        cx2 = plsc.load_gather(x2v, [liv])
        cy2 = plsc.load_gather(y2v, [liv])
        car = plsc.load_gather(arv, [liv])
        clb = plsc.load_gather(labv, [liv])

        pub = jnp.full((L,), Ml, jnp.float32)
        pub = jnp.where(lanes == 1,
                        plsc.bitcast(jnp.full((L,), il, jnp.int32),
                                     jnp.float32), pub)
        pub = jnp.where(lanes == 2, cx1, pub)
        pub = jnp.where(lanes == 3, cy1, pub)
        pub = jnp.where(lanes == 4, cx2, pub)
        pub = jnp.where(lanes == 5, cy2, pub)
        pub = jnp.where(lanes == 6, plsc.bitcast(clb, jnp.float32), pub)
        pub = jnp.where(lanes == 7, car, pub)
        pubv[...] = pub
        pltpu.sync_copy(pubv, shared.at[pl.ds(sid * L, L)])
        plsc.subcore_barrier()
        pltpu.sync_copy(shared, rbv)
        plsc.subcore_barrier()

        def col(c):
            return plsc.load_gather(rbv, [lanes * L + c])

        s_all = col(0)
        i_all = plsc.bitcast(col(1), jnp.int32)
        M = jnp.max(s_all)
        cont2 = M > -1e29
        iw = jnp.min(jnp.where(s_all == M, i_all,
                               jnp.full((L,), BIG, jnp.int32)))
        wm = i_all == iw
        negv = jnp.full((L,), NEG, jnp.float32)
        X1 = jnp.max(jnp.where(wm, col(2), negv))
        Y1 = jnp.max(jnp.where(wm, col(3), negv))
        X2 = jnp.max(jnp.where(wm, col(4), negv))
        Y2 = jnp.max(jnp.where(wm, col(5), negv))
        LB = jnp.max(jnp.where(wm, plsc.bitcast(col(6), jnp.int32),
                               jnp.full((L,), -1, jnp.int32)))
        WA = jnp.max(jnp.where(wm, col(7), negv))

        X1v = jnp.full((L,), X1, jnp.float32)
        Y1v = jnp.full((L,), Y1, jnp.float32)
        X2v = jnp.full((L,), X2, jnp.float32)
        Y2v = jnp.full((L,), Y2, jnp.float32)
        WAv = jnp.full((L,), WA, jnp.float32)
        iwv = jnp.full((L,), iw, jnp.int32)

        @pl.when(cont2)
        def _():
            def sup_j(j, _):
                sl = pl.ds(j * L, L)
                xx1 = jnp.maximum(x1v[sl], X1v)
                yy1 = jnp.maximum(y1v[sl], Y1v)
                xx2 = jnp.minimum(x2v[sl], X2v)
                yy2 = jnp.minimum(y2v[sl], Y2v)
                inter = (jnp.maximum(xx2 - xx1, 0.0) *
                         jnp.maximum(yy2 - yy1, 0.0))
                union = WAv + arv[sl] - inter
                iou = inter / (union + 1e-8)
                gi = base + j * L + lanes
                kill = (iou >= IOU_THR) | (gi == iwv)
                candv[sl] = jnp.where(kill, negv, candv[sl])
                return 0
            lax.fori_loop(0, NV, sup_j, 0)

        @pl.when(cont2 & (sid == 0))
        def _():
            bvals = X1v
            bvals = jnp.where(lanes == 1, Y1v, bvals)
            bvals = jnp.where(lanes == 2, X2v, bvals)
            bvals = jnp.where(lanes == 3, Y2v, bvals)
            plsc.store_scatter(obv, [4 * k + lanes], bvals, mask=lanes < 4)
            kv = jnp.full((L,), k, jnp.int32)
            plsc.store_scatter(osv, [kv], jnp.full((L,), M, jnp.float32),
                               mask=lanes == 0)
            plsc.store_scatter(olv, [kv], jnp.full((L,), LB, jnp.int32),
                               mask=lanes == 0)

        k2 = k + cont2.astype(jnp.int32)
        return (k2, cont2 & (k2 < K))

    lax.while_loop(cond, body, (jnp.int32(0), jnp.bool_(True)))

    @pl.when(sid == 0)
    def _():
        pltpu.sync_copy(obv, bo_hbm)
        pltpu.sync_copy(osv, so_hbm)
        pltpu.sync_copy(olv, lo_hbm)


@functools.partial(
    pl.kernel,
    out_type=(
        jax.ShapeDtypeStruct((BO_PAD,), jnp.float32),
        jax.ShapeDtypeStruct((SC_PAD,), jnp.float32),
        jax.ShapeDtypeStruct((SC_PAD,), jnp.int32),
    ),
    mesh=plsc.VectorSubcoreMesh(
        core_axis_name="c", subcore_axis_name="s",
        num_cores=1, num_subcores=NS),
    compiler_params=pltpu.CompilerParams(needs_layout_passes=False),
    scratch_types=[
        pltpu.VMEM((ROWS * C,), jnp.float32),
        pltpu.VMEM((ROWS,), jnp.float32),
        pltpu.VMEM((ROWS,), jnp.float32),
        pltpu.VMEM((ROWS,), jnp.float32),
        pltpu.VMEM((ROWS,), jnp.float32),
        pltpu.VMEM((ROWS,), jnp.float32),
        pltpu.VMEM((ROWS,), jnp.float32),
        pltpu.VMEM((ROWS,), jnp.int32),
        pltpu.VMEM((L,), jnp.float32),
        pltpu.VMEM((NS * L,), jnp.float32),
        pltpu.VMEM((BO_PAD,), jnp.float32),
        pltpu.VMEM((SC_PAD,), jnp.float32),
        pltpu.VMEM((SC_PAD,), jnp.int32),
        pltpu.VMEM_SHARED((NS * L,), jnp.float32),
    ],
)
def _nms_call(sc_hbm, x1_hbm, y1_hbm, x2_hbm, y2_hbm,
              bo_hbm, so_hbm, lo_hbm, *scratch):
    _nms_kernel(sc_hbm, x1_hbm, y1_hbm, x2_hbm, y2_hbm,
                bo_hbm, so_hbm, lo_hbm, *scratch)


@jax.jit
def kernel(boxes, scores):
    bp = jnp.pad(boxes, ((0, NP - N), (0, 0)))
    sp = jnp.pad(scores, ((0, NP - N), (0, 0)))
    bo, so, lo = _nms_call(
        sp.reshape(-1),
        bp[:, 0], bp[:, 1], bp[:, 2], bp[:, 3])
    return (bo[:4 * K].reshape(1, K, 4), so[:K][None], lo[:K][None])
